# baseline (speedup 1.0000x reference)
import functools

import jax
import jax.numpy as jnp
from jax import lax
from jax.experimental import pallas as pl
from jax.experimental.pallas import tpu as pltpu
from jax.experimental.pallas import tpu_sc as plsc

_E = 8
_T = 4096
_D = 1024
_H = 512
_BT = 512
_BTG = 128
_NBP = 72
_NROWS = _NBP * _BTG
_NW = 32
_TPW = _T // _NW
_CD = 64
_CC = 32
_D2 = _D // 2


def _router_body(x_ref, sw_ref, tril_ref, gate_ref, krank_ref, grank_ref,
                 counts_ref, xbf_ref, carry):
    i = pl.program_id(0)

    @pl.when(i == 0)
    def _init():
        carry[...] = jnp.zeros_like(carry)

    xl = x_ref[:, :_D2].astype(jnp.bfloat16).astype(jnp.float32)
    xh = x_ref[:, _D2:].astype(jnp.bfloat16).astype(jnp.float32)
    ul = jax.lax.bitcast_convert_type(xl, jnp.uint32) >> 16
    uh = jax.lax.bitcast_convert_type(xh, jnp.uint32) & jnp.uint32(0xFFFF0000)
    xbf_ref[...] = jax.lax.bitcast_convert_type(ul | uh, jnp.int32)
    logits = jax.lax.dot_general(
        x_ref[...], sw_ref[...], (((1,), (1,)), ((), ())),
        preferred_element_type=jnp.float32)
    eidx = jax.lax.broadcasted_iota(jnp.int32, logits.shape, 1)
    krank = jnp.zeros_like(logits)
    for j in range(_E):
        vj = logits[:, j:j + 1]
        beats = (vj > logits) | ((vj == logits) & (j < eidx))
        krank += beats.astype(jnp.float32)
    gate_ref[...] = jnp.where(krank < 2.0, jax.nn.sigmoid(logits), 0.0)
    krank_ref[...] = krank

    ind = (krank < 2.0).astype(jnp.float32)
    local = jax.lax.dot_general(
        tril_ref[...], ind, (((1,), (0,)), ((), ())),
        preferred_element_type=jnp.float32)
    grank_ref[...] = local + carry[...]
    carry[...] += jnp.sum(ind, axis=0, keepdims=True)
    counts_ref[...] = carry[...]


def _prep_body(gate_ref, krank_ref, grank_ref, counts_ref, triu_ref,
               dsta_ref, dstb_ref, ga_ref, gb_ref, blk_ref):
    counts = counts_ref[...]
    pc = jnp.floor((counts + (_BTG - 1.0)) / _BTG)
    offb = jax.lax.dot_general(
        pc, triu_ref[...], (((1,), (0,)), ((), ())),
        preferred_element_type=jnp.float32)

    gate = gate_ref[...]
    krank = krank_ref[...]
    grank = grank_ref[...]
    sel = krank < 2.0
    eidx = jax.lax.broadcasted_iota(jnp.int32, gate.shape, 1).astype(
        jnp.float32)
    e_a = jnp.min(jnp.where(sel, eidx, 99.0), axis=1, keepdims=True)
    e_b = jnp.max(jnp.where(sel, eidx, -1.0), axis=1, keepdims=True)
    m_a = (eidx == e_a).astype(jnp.float32)
    m_b = (eidx == e_b).astype(jnp.float32)
    ga_ref[...] = jnp.sum(gate * m_a, axis=1, keepdims=True)
    gb_ref[...] = jnp.sum(gate * m_b, axis=1, keepdims=True)
    off_a = jnp.sum(offb * m_a, axis=1, keepdims=True) * _BTG
    off_b = jnp.sum(offb * m_b, axis=1, keepdims=True) * _BTG
    r_a = jnp.sum(grank * m_a, axis=1, keepdims=True)
    r_b = jnp.sum(grank * m_b, axis=1, keepdims=True)
    dsta_ref[...] = (off_a + r_a).astype(jnp.int32)
    dstb_ref[...] = (off_b + r_b).astype(jnp.int32)

    b_io = jax.lax.broadcasted_iota(jnp.int32, (_NBP, 1), 0).astype(
        jnp.float32)
    cnt = jnp.sum((b_io >= offb).astype(jnp.float32), axis=1, keepdims=True)
    blk_ref[...] = (cnt - 1.0).astype(jnp.int32)


@functools.lru_cache(maxsize=1)
def _make_sc_dispatch():
    @functools.partial(
        pl.kernel,
        mesh=plsc.VectorSubcoreMesh(core_axis_name="c", subcore_axis_name="s"),
        out_type=jax.ShapeDtypeStruct((_NROWS, _D2), jnp.int32),
        scratch_types=[
            pltpu.VMEM((_CD, _D2), jnp.int32),
            pltpu.VMEM((_CD, _D2), jnp.int32),
            pltpu.VMEM((_CD,), jnp.int32),
            pltpu.VMEM((_CD,), jnp.int32),
            pltpu.VMEM((_CD,), jnp.int32),
            pltpu.VMEM((_CD,), jnp.int32),
            pltpu.SemaphoreType.DMA,
            pltpu.SemaphoreType.DMA,
        ],
    )
    def dispatch(x_hbm, dsta_hbm, dstb_hbm, xs_hbm, rows0, rows1, ia0, ib0,
                 ia1, ib1, sem_l, sem_s):
        rows = (rows0, rows1)
        ia = (ia0, ia1)
        ib = (ib0, ib1)
        wid = lax.axis_index("s") * 2 + lax.axis_index("c")
        base = wid * _TPW
        nch = _TPW // _CD
        pltpu.sync_copy(dsta_hbm.at[pl.ds(base, _CD)], ia0)
        pltpu.sync_copy(dstb_hbm.at[pl.ds(base, _CD)], ib0)
        ld = pltpu.async_copy(x_hbm.at[pl.ds(base, _CD)], rows0, sem_l)
        pend = [None, None]
        for c in range(nch):
            p = c % 2
            q = 1 - p
            ld.wait()
            sa = pltpu.async_copy(rows[p], xs_hbm.at[ia[p]], sem_s)
            sb = pltpu.async_copy(rows[p], xs_hbm.at[ib[p]], sem_s)
            pend[p] = (sa, sb)
            if c + 1 < nch:
                if pend[q] is not None:
                    pend[q][0].wait()
                    pend[q][1].wait()
                    pend[q] = None
                off1 = base + (c + 1) * _CD
                pltpu.sync_copy(dsta_hbm.at[pl.ds(off1, _CD)], ia[q])
                pltpu.sync_copy(dstb_hbm.at[pl.ds(off1, _CD)], ib[q])
                ld = pltpu.async_copy(x_hbm.at[pl.ds(off1, _CD)], rows[q],
                                      sem_l)
        for pp in pend:
            if pp is not None:
                pp[0].wait()
                pp[1].wait()

    return dispatch


def _sc_dispatch(xpk, dsta, dstb):
    return _make_sc_dispatch()(xpk, dsta, dstb)


def _unpack_halves(pk):
    u = jax.lax.bitcast_convert_type(pk, jnp.uint32)
    lo = jax.lax.bitcast_convert_type(u << 16, jnp.float32)
    hi = jax.lax.bitcast_convert_type(u & jnp.uint32(0xFFFF0000), jnp.float32)
    return lo, hi


def _pack_halves(lo, hi):
    ul = jax.lax.bitcast_convert_type(
        lo.astype(jnp.bfloat16).astype(jnp.float32), jnp.uint32) >> 16
    uh = jax.lax.bitcast_convert_type(
        hi.astype(jnp.bfloat16).astype(jnp.float32),
        jnp.uint32) & jnp.uint32(0xFFFF0000)
    return jax.lax.bitcast_convert_type(ul | uh, jnp.int32)


def _group_body(blk_ref, xs_ref, w1_ref, b1_ref, w2_ref, b2_ref, ys_ref):
    xlo, xhi = _unpack_halves(xs_ref[...])
    xb = jnp.concatenate([xlo, xhi], axis=1).astype(jnp.bfloat16)
    h = jax.lax.dot_general(
        xb, w1_ref[0].astype(jnp.bfloat16), (((1,), (1,)), ((), ())),
        preferred_element_type=jnp.float32)
    h = jnp.maximum(h + b1_ref[0], 0.0).astype(jnp.bfloat16)
    y = jax.lax.dot_general(
        h, w2_ref[0].astype(jnp.bfloat16), (((1,), (1,)), ((), ())),
        preferred_element_type=jnp.float32)
    y = y + b2_ref[0]
    ys_ref[...] = _pack_halves(y[:, :_D2], y[:, _D2:])


@functools.lru_cache(maxsize=1)
def _make_sc_combine():
    @functools.partial(
        pl.kernel,
        mesh=plsc.VectorSubcoreMesh(core_axis_name="c", subcore_axis_name="s"),
        out_type=[
            jax.ShapeDtypeStruct((_T, _D2), jnp.int32),
            jax.ShapeDtypeStruct((_T, _D2), jnp.int32),
        ],
        scratch_types=[
            pltpu.VMEM((_CC, _D2), jnp.int32),
            pltpu.VMEM((_CC, _D2), jnp.int32),
            pltpu.VMEM((_CC, _D2), jnp.int32),
            pltpu.VMEM((_CC, _D2), jnp.int32),
            pltpu.VMEM((_CC,), jnp.int32),
            pltpu.VMEM((_CC,), jnp.int32),
            pltpu.VMEM((_CC,), jnp.int32),
            pltpu.VMEM((_CC,), jnp.int32),
            pltpu.SemaphoreType.DMA,
            pltpu.SemaphoreType.DMA,
        ],
    )
    def combine(ys_hbm, dsta_hbm, dstb_hbm, a_hbm, b_hbm, ra0, rb0, ra1, rb1,
                ia0, ib0, ia1, ib1, sem_g, sem_s):
        ra = (ra0, ra1)
        rb = (rb0, rb1)
        ia = (ia0, ia1)
        ib = (ib0, ib1)
        wid = lax.axis_index("s") * 2 + lax.axis_index("c")
        base = wid * _TPW
        nch = _TPW // _CC
        pltpu.sync_copy(dsta_hbm.at[pl.ds(base, _CC)], ia0)
        pltpu.sync_copy(dstb_hbm.at[pl.ds(base, _CC)], ib0)
        ga = pltpu.async_copy(ys_hbm.at[ia0], ra0, sem_g)
        gb = pltpu.async_copy(ys_hbm.at[ib0], rb0, sem_g)
        pend = [None, None]
        for c in range(nch):
            p = c % 2
            q = 1 - p
            off = base + c * _CC
            ga.wait()
            gb.wait()
            sa = pltpu.async_copy(ra[p], a_hbm.at[pl.ds(off, _CC)], sem_s)
            sb = pltpu.async_copy(rb[p], b_hbm.at[pl.ds(off, _CC)], sem_s)
            pend[p] = (sa, sb)
            if c + 1 < nch:
                if pend[q] is not None:
                    pend[q][0].wait()
                    pend[q][1].wait()
                    pend[q] = None
                off1 = base + (c + 1) * _CC
                pltpu.sync_copy(dsta_hbm.at[pl.ds(off1, _CC)], ia[q])
                pltpu.sync_copy(dstb_hbm.at[pl.ds(off1, _CC)], ib[q])
                ga = pltpu.async_copy(ys_hbm.at[ia[q]], ra[q], sem_g)
                gb = pltpu.async_copy(ys_hbm.at[ib[q]], rb[q], sem_g)
        for pp in pend:
            if pp is not None:
                pp[0].wait()
                pp[1].wait()

    return combine


def _sc_combine(ys, dsta, dstb):
    return _make_sc_combine()(ys, dsta, dstb)


def _add_body(a_ref, b_ref, ga_ref, gb_ref, o_ref):
    alo, ahi = _unpack_halves(a_ref[...])
    blo, bhi = _unpack_halves(b_ref[...])
    ga = ga_ref[...]
    gb = gb_ref[...]
    o_ref[...] = jnp.concatenate(
        [ga * alo + gb * blo, ga * ahi + gb * bhi], axis=1)


def _run_add(a, b, ga, gb):
    return pl.pallas_call(
        _add_body,
        grid=(_T // _BT,),
        in_specs=[
            pl.BlockSpec((_BT, _D2), lambda i: (i, 0)),
            pl.BlockSpec((_BT, _D2), lambda i: (i, 0)),
            pl.BlockSpec((_BT, 1), lambda i: (i, 0)),
            pl.BlockSpec((_BT, 1), lambda i: (i, 0)),
        ],
        out_specs=pl.BlockSpec((_BT, _D), lambda i: (i, 0)),
        out_shape=jax.ShapeDtypeStruct((_T, _D), jnp.float32),
    )(a, b, ga, gb)


def _run_router(xt, expert_sel):
    r = jax.lax.broadcasted_iota(jnp.int32, (_BT, _BT), 0)
    c = jax.lax.broadcasted_iota(jnp.int32, (_BT, _BT), 1)
    tril = (c < r).astype(jnp.float32)
    return pl.pallas_call(
        _router_body,
        grid=(_T // _BT,),
        in_specs=[
            pl.BlockSpec((_BT, _D), lambda i: (i, 0)),
            pl.BlockSpec((_E, _D), lambda i: (0, 0)),
            pl.BlockSpec((_BT, _BT), lambda i: (0, 0)),
        ],
        out_specs=[
            pl.BlockSpec((_BT, _E), lambda i: (i, 0)),
            pl.BlockSpec((_BT, _E), lambda i: (i, 0)),
            pl.BlockSpec((_BT, _E), lambda i: (i, 0)),
            pl.BlockSpec((1, _E), lambda i: (0, 0)),
            pl.BlockSpec((_BT, _D2), lambda i: (i, 0)),
        ],
        out_shape=[
            jax.ShapeDtypeStruct((_T, _E), jnp.float32),
            jax.ShapeDtypeStruct((_T, _E), jnp.float32),
            jax.ShapeDtypeStruct((_T, _E), jnp.float32),
            jax.ShapeDtypeStruct((1, _E), jnp.float32),
            jax.ShapeDtypeStruct((_T, _D2), jnp.int32),
        ],
        scratch_shapes=[pltpu.VMEM((1, _E), jnp.float32)],
    )(xt, expert_sel, tril)


def _run_prep(gate, krank, grank, counts):
    r = jax.lax.broadcasted_iota(jnp.int32, (_E, _E), 0)
    c = jax.lax.broadcasted_iota(jnp.int32, (_E, _E), 1)
    triu = (r < c).astype(jnp.float32)
    return pl.pallas_call(
        _prep_body,
        grid=(_T // _BT,),
        in_specs=[
            pl.BlockSpec((_BT, _E), lambda i: (i, 0)),
            pl.BlockSpec((_BT, _E), lambda i: (i, 0)),
            pl.BlockSpec((_BT, _E), lambda i: (i, 0)),
            pl.BlockSpec((1, _E), lambda i: (0, 0)),
            pl.BlockSpec((_E, _E), lambda i: (0, 0)),
        ],
        out_specs=[
            pl.BlockSpec((_BT, 1), lambda i: (i, 0)),
            pl.BlockSpec((_BT, 1), lambda i: (i, 0)),
            pl.BlockSpec((_BT, 1), lambda i: (i, 0)),
            pl.BlockSpec((_BT, 1), lambda i: (i, 0)),
            pl.BlockSpec((_NBP, 1), lambda i: (0, 0)),
        ],
        out_shape=[
            jax.ShapeDtypeStruct((_T, 1), jnp.int32),
            jax.ShapeDtypeStruct((_T, 1), jnp.int32),
            jax.ShapeDtypeStruct((_T, 1), jnp.float32),
            jax.ShapeDtypeStruct((_T, 1), jnp.float32),
            jax.ShapeDtypeStruct((_NBP, 1), jnp.int32),
        ],
    )(gate, krank, grank, counts, triu)


def _run_group(blkmap, xs, w1, b1, w2, b2):
    grid_spec = pltpu.PrefetchScalarGridSpec(
        num_scalar_prefetch=1,
        grid=(_NBP,),
        in_specs=[
            pl.BlockSpec((_BTG, _D2), lambda b, m: (b, 0)),
            pl.BlockSpec((1, _H, _D), lambda b, m: (m[b], 0, 0)),
            pl.BlockSpec((1, 1, _H), lambda b, m: (m[b], 0, 0)),
            pl.BlockSpec((1, _D, _H), lambda b, m: (m[b], 0, 0)),
            pl.BlockSpec((1, 1, _D), lambda b, m: (m[b], 0, 0)),
        ],
        out_specs=pl.BlockSpec((_BTG, _D2), lambda b, m: (b, 0)),
    )
    return pl.pallas_call(
        _group_body,
        grid_spec=grid_spec,
        out_shape=jax.ShapeDtypeStruct((_NROWS, _D2), jnp.int32),
    )(blkmap, xs, w1, b1.reshape(_E, 1, _H), w2, b2.reshape(_E, 1, _D))


def kernel(x, expert_sel, w1, b1, w2, b2):
    B, S, D = x.shape
    xt = x.reshape(B * S, D)
    gate, krank, grank, counts, xbf = _run_router(xt, expert_sel)
    dsta, dstb, ga, gb, blkmap = _run_prep(gate, krank, grank, counts)
    xs = _sc_dispatch(xbf, dsta.reshape(_T), dstb.reshape(_T))
    ys = _run_group(blkmap.reshape(_NBP), xs, w1, b1, w2, b2)
    ya, yb = _sc_combine(ys, dsta.reshape(_T), dstb.reshape(_T))
    out = _run_add(ya, yb, ga, gb)
    return out.reshape(B, S, D), jnp.array(0.0, dtype=x.dtype)


# speedup vs baseline: 1.2079x; 1.2079x over previous
import functools

import jax
import jax.numpy as jnp
from jax import lax
from jax.experimental import pallas as pl
from jax.experimental.pallas import tpu as pltpu
from jax.experimental.pallas import tpu_sc as plsc

_E = 8
_T = 4096
_D = 1024
_H = 512
_BT = 512
_BTG = 256
_NBP = 40
_NROWS = _NBP * _BTG
_NW = 32
_TPW = _T // _NW
_CD = 64
_CC = 32
_D2 = _D // 2


def _router_body(x_ref, sw_ref, tril_ref, gate_ref, krank_ref, grank_ref,
                 counts_ref, xbf_ref, carry):
    i = pl.program_id(0)

    @pl.when(i == 0)
    def _init():
        carry[...] = jnp.zeros_like(carry)

    xl = x_ref[:, :_D2].astype(jnp.bfloat16).astype(jnp.float32)
    xh = x_ref[:, _D2:].astype(jnp.bfloat16).astype(jnp.float32)
    ul = jax.lax.bitcast_convert_type(xl, jnp.uint32) >> 16
    uh = jax.lax.bitcast_convert_type(xh, jnp.uint32) & jnp.uint32(0xFFFF0000)
    xbf_ref[...] = jax.lax.bitcast_convert_type(ul | uh, jnp.int32)
    logits = jax.lax.dot_general(
        x_ref[...], sw_ref[...], (((1,), (1,)), ((), ())),
        preferred_element_type=jnp.float32)
    eidx = jax.lax.broadcasted_iota(jnp.int32, logits.shape, 1)
    krank = jnp.zeros_like(logits)
    for j in range(_E):
        vj = logits[:, j:j + 1]
        beats = (vj > logits) | ((vj == logits) & (j < eidx))
        krank += beats.astype(jnp.float32)
    gate_ref[...] = jnp.where(krank < 2.0, jax.nn.sigmoid(logits), 0.0)
    krank_ref[...] = krank

    ind = (krank < 2.0).astype(jnp.float32)
    local = jax.lax.dot_general(
        tril_ref[...], ind, (((1,), (0,)), ((), ())),
        preferred_element_type=jnp.float32)
    grank_ref[...] = local + carry[...]
    carry[...] += jnp.sum(ind, axis=0, keepdims=True)
    counts_ref[...] = carry[...]


def _prep_body(gate_ref, krank_ref, grank_ref, counts_ref, triu_ref,
               dsta_ref, dstb_ref, ga_ref, gb_ref, blk_ref):
    counts = counts_ref[...]
    pc = jnp.floor((counts + (_BTG - 1.0)) / _BTG)
    offb = jax.lax.dot_general(
        pc, triu_ref[...], (((1,), (0,)), ((), ())),
        preferred_element_type=jnp.float32)

    gate = gate_ref[...]
    krank = krank_ref[...]
    grank = grank_ref[...]
    sel = krank < 2.0
    eidx = jax.lax.broadcasted_iota(jnp.int32, gate.shape, 1).astype(
        jnp.float32)
    e_a = jnp.min(jnp.where(sel, eidx, 99.0), axis=1, keepdims=True)
    e_b = jnp.max(jnp.where(sel, eidx, -1.0), axis=1, keepdims=True)
    m_a = (eidx == e_a).astype(jnp.float32)
    m_b = (eidx == e_b).astype(jnp.float32)
    ga_ref[...] = jnp.sum(gate * m_a, axis=1, keepdims=True)
    gb_ref[...] = jnp.sum(gate * m_b, axis=1, keepdims=True)
    off_a = jnp.sum(offb * m_a, axis=1, keepdims=True) * _BTG
    off_b = jnp.sum(offb * m_b, axis=1, keepdims=True) * _BTG
    r_a = jnp.sum(grank * m_a, axis=1, keepdims=True)
    r_b = jnp.sum(grank * m_b, axis=1, keepdims=True)
    dsta_ref[...] = (off_a + r_a).astype(jnp.int32)
    dstb_ref[...] = (off_b + r_b).astype(jnp.int32)

    b_io = jax.lax.broadcasted_iota(jnp.int32, (_NBP, 1), 0).astype(
        jnp.float32)
    cnt = jnp.sum((b_io >= offb).astype(jnp.float32), axis=1, keepdims=True)
    blk_ref[...] = (cnt - 1.0).astype(jnp.int32)


@functools.lru_cache(maxsize=1)
def _make_sc_dispatch():
    @functools.partial(
        pl.kernel,
        mesh=plsc.VectorSubcoreMesh(core_axis_name="c", subcore_axis_name="s"),
        out_type=jax.ShapeDtypeStruct((_NROWS, _D2), jnp.int32),
        scratch_types=[
            pltpu.VMEM((_CD, _D2), jnp.int32),
            pltpu.VMEM((_CD, _D2), jnp.int32),
            pltpu.VMEM((_CD,), jnp.int32),
            pltpu.VMEM((_CD,), jnp.int32),
            pltpu.VMEM((_CD,), jnp.int32),
            pltpu.VMEM((_CD,), jnp.int32),
            pltpu.SemaphoreType.DMA,
            pltpu.SemaphoreType.DMA,
        ],
    )
    def dispatch(x_hbm, dsta_hbm, dstb_hbm, xs_hbm, rows0, rows1, ia0, ib0,
                 ia1, ib1, sem_l, sem_s):
        rows = (rows0, rows1)
        ia = (ia0, ia1)
        ib = (ib0, ib1)
        wid = lax.axis_index("s") * 2 + lax.axis_index("c")
        base = wid * _TPW
        nch = _TPW // _CD
        pltpu.sync_copy(dsta_hbm.at[pl.ds(base, _CD)], ia0)
        pltpu.sync_copy(dstb_hbm.at[pl.ds(base, _CD)], ib0)
        ld = pltpu.async_copy(x_hbm.at[pl.ds(base, _CD)], rows0, sem_l)
        pend = [None, None]
        for c in range(nch):
            p = c % 2
            q = 1 - p
            ld.wait()
            sa = pltpu.async_copy(rows[p], xs_hbm.at[ia[p]], sem_s)
            sb = pltpu.async_copy(rows[p], xs_hbm.at[ib[p]], sem_s)
            pend[p] = (sa, sb)
            if c + 1 < nch:
                if pend[q] is not None:
                    pend[q][0].wait()
                    pend[q][1].wait()
                    pend[q] = None
                off1 = base + (c + 1) * _CD
                pltpu.sync_copy(dsta_hbm.at[pl.ds(off1, _CD)], ia[q])
                pltpu.sync_copy(dstb_hbm.at[pl.ds(off1, _CD)], ib[q])
                ld = pltpu.async_copy(x_hbm.at[pl.ds(off1, _CD)], rows[q],
                                      sem_l)
        for pp in pend:
            if pp is not None:
                pp[0].wait()
                pp[1].wait()

    return dispatch


def _sc_dispatch(xpk, dsta, dstb):
    return _make_sc_dispatch()(xpk, dsta, dstb)


def _unpack_halves(pk):
    u = jax.lax.bitcast_convert_type(pk, jnp.uint32)
    lo = jax.lax.bitcast_convert_type(u << 16, jnp.float32)
    hi = jax.lax.bitcast_convert_type(u & jnp.uint32(0xFFFF0000), jnp.float32)
    return lo, hi


def _pack_halves(lo, hi):
    ul = jax.lax.bitcast_convert_type(
        lo.astype(jnp.bfloat16).astype(jnp.float32), jnp.uint32) >> 16
    uh = jax.lax.bitcast_convert_type(
        hi.astype(jnp.bfloat16).astype(jnp.float32),
        jnp.uint32) & jnp.uint32(0xFFFF0000)
    return jax.lax.bitcast_convert_type(ul | uh, jnp.int32)


def _group_body(blk_ref, xs_ref, w1_ref, b1_ref, w2_ref, b2_ref, ys_ref):
    xlo, xhi = _unpack_halves(xs_ref[...])
    xb = jnp.concatenate([xlo, xhi], axis=1).astype(jnp.bfloat16)
    h = jax.lax.dot_general(
        xb, w1_ref[0].astype(jnp.bfloat16), (((1,), (1,)), ((), ())),
        preferred_element_type=jnp.float32)
    h = jnp.maximum(h + b1_ref[0], 0.0).astype(jnp.bfloat16)
    y = jax.lax.dot_general(
        h, w2_ref[0].astype(jnp.bfloat16), (((1,), (1,)), ((), ())),
        preferred_element_type=jnp.float32)
    y = y + b2_ref[0]
    ys_ref[...] = _pack_halves(y[:, :_D2], y[:, _D2:])


@functools.lru_cache(maxsize=1)
def _make_sc_combine():
    @functools.partial(
        pl.kernel,
        mesh=plsc.VectorSubcoreMesh(core_axis_name="c", subcore_axis_name="s"),
        out_type=[
            jax.ShapeDtypeStruct((_T, _D2), jnp.int32),
            jax.ShapeDtypeStruct((_T, _D2), jnp.int32),
        ],
        scratch_types=[
            pltpu.VMEM((_CC, _D2), jnp.int32),
            pltpu.VMEM((_CC, _D2), jnp.int32),
            pltpu.VMEM((_CC, _D2), jnp.int32),
            pltpu.VMEM((_CC, _D2), jnp.int32),
            pltpu.VMEM((_CC,), jnp.int32),
            pltpu.VMEM((_CC,), jnp.int32),
            pltpu.VMEM((_CC,), jnp.int32),
            pltpu.VMEM((_CC,), jnp.int32),
            pltpu.SemaphoreType.DMA,
            pltpu.SemaphoreType.DMA,
        ],
    )
    def combine(ys_hbm, dsta_hbm, dstb_hbm, a_hbm, b_hbm, ra0, rb0, ra1, rb1,
                ia0, ib0, ia1, ib1, sem_g, sem_s):
        ra = (ra0, ra1)
        rb = (rb0, rb1)
        ia = (ia0, ia1)
        ib = (ib0, ib1)
        wid = lax.axis_index("s") * 2 + lax.axis_index("c")
        base = wid * _TPW
        nch = _TPW // _CC
        pltpu.sync_copy(dsta_hbm.at[pl.ds(base, _CC)], ia0)
        pltpu.sync_copy(dstb_hbm.at[pl.ds(base, _CC)], ib0)
        ga = pltpu.async_copy(ys_hbm.at[ia0], ra0, sem_g)
        gb = pltpu.async_copy(ys_hbm.at[ib0], rb0, sem_g)
        pend = [None, None]
        for c in range(nch):
            p = c % 2
            q = 1 - p
            off = base + c * _CC
            ga.wait()
            gb.wait()
            sa = pltpu.async_copy(ra[p], a_hbm.at[pl.ds(off, _CC)], sem_s)
            sb = pltpu.async_copy(rb[p], b_hbm.at[pl.ds(off, _CC)], sem_s)
            pend[p] = (sa, sb)
            if c + 1 < nch:
                if pend[q] is not None:
                    pend[q][0].wait()
                    pend[q][1].wait()
                    pend[q] = None
                off1 = base + (c + 1) * _CC
                pltpu.sync_copy(dsta_hbm.at[pl.ds(off1, _CC)], ia[q])
                pltpu.sync_copy(dstb_hbm.at[pl.ds(off1, _CC)], ib[q])
                ga = pltpu.async_copy(ys_hbm.at[ia[q]], ra[q], sem_g)
                gb = pltpu.async_copy(ys_hbm.at[ib[q]], rb[q], sem_g)
        for pp in pend:
            if pp is not None:
                pp[0].wait()
                pp[1].wait()

    return combine


def _sc_combine(ys, dsta, dstb):
    return _make_sc_combine()(ys, dsta, dstb)


def _add_body(a_ref, b_ref, ga_ref, gb_ref, o_ref):
    alo, ahi = _unpack_halves(a_ref[...])
    blo, bhi = _unpack_halves(b_ref[...])
    ga = ga_ref[...]
    gb = gb_ref[...]
    o_ref[...] = jnp.concatenate(
        [ga * alo + gb * blo, ga * ahi + gb * bhi], axis=1)


def _run_add(a, b, ga, gb):
    return pl.pallas_call(
        _add_body,
        grid=(_T // _BT,),
        in_specs=[
            pl.BlockSpec((_BT, _D2), lambda i: (i, 0)),
            pl.BlockSpec((_BT, _D2), lambda i: (i, 0)),
            pl.BlockSpec((_BT, 1), lambda i: (i, 0)),
            pl.BlockSpec((_BT, 1), lambda i: (i, 0)),
        ],
        out_specs=pl.BlockSpec((_BT, _D), lambda i: (i, 0)),
        out_shape=jax.ShapeDtypeStruct((_T, _D), jnp.float32),
    )(a, b, ga, gb)


def _run_router(xt, expert_sel):
    r = jax.lax.broadcasted_iota(jnp.int32, (_BT, _BT), 0)
    c = jax.lax.broadcasted_iota(jnp.int32, (_BT, _BT), 1)
    tril = (c < r).astype(jnp.float32)
    return pl.pallas_call(
        _router_body,
        grid=(_T // _BT,),
        in_specs=[
            pl.BlockSpec((_BT, _D), lambda i: (i, 0)),
            pl.BlockSpec((_E, _D), lambda i: (0, 0)),
            pl.BlockSpec((_BT, _BT), lambda i: (0, 0)),
        ],
        out_specs=[
            pl.BlockSpec((_BT, _E), lambda i: (i, 0)),
            pl.BlockSpec((_BT, _E), lambda i: (i, 0)),
            pl.BlockSpec((_BT, _E), lambda i: (i, 0)),
            pl.BlockSpec((1, _E), lambda i: (0, 0)),
            pl.BlockSpec((_BT, _D2), lambda i: (i, 0)),
        ],
        out_shape=[
            jax.ShapeDtypeStruct((_T, _E), jnp.float32),
            jax.ShapeDtypeStruct((_T, _E), jnp.float32),
            jax.ShapeDtypeStruct((_T, _E), jnp.float32),
            jax.ShapeDtypeStruct((1, _E), jnp.float32),
            jax.ShapeDtypeStruct((_T, _D2), jnp.int32),
        ],
        scratch_shapes=[pltpu.VMEM((1, _E), jnp.float32)],
    )(xt, expert_sel, tril)


def _run_prep(gate, krank, grank, counts):
    r = jax.lax.broadcasted_iota(jnp.int32, (_E, _E), 0)
    c = jax.lax.broadcasted_iota(jnp.int32, (_E, _E), 1)
    triu = (r < c).astype(jnp.float32)
    return pl.pallas_call(
        _prep_body,
        grid=(1,),
        in_specs=[
            pl.BlockSpec((_T, _E), lambda i: (0, 0)),
            pl.BlockSpec((_T, _E), lambda i: (0, 0)),
            pl.BlockSpec((_T, _E), lambda i: (0, 0)),
            pl.BlockSpec((1, _E), lambda i: (0, 0)),
            pl.BlockSpec((_E, _E), lambda i: (0, 0)),
        ],
        out_specs=[
            pl.BlockSpec((_T, 1), lambda i: (0, 0)),
            pl.BlockSpec((_T, 1), lambda i: (0, 0)),
            pl.BlockSpec((_T, 1), lambda i: (0, 0)),
            pl.BlockSpec((_T, 1), lambda i: (0, 0)),
            pl.BlockSpec((_NBP, 1), lambda i: (0, 0)),
        ],
        out_shape=[
            jax.ShapeDtypeStruct((_T, 1), jnp.int32),
            jax.ShapeDtypeStruct((_T, 1), jnp.int32),
            jax.ShapeDtypeStruct((_T, 1), jnp.float32),
            jax.ShapeDtypeStruct((_T, 1), jnp.float32),
            jax.ShapeDtypeStruct((_NBP, 1), jnp.int32),
        ],
    )(gate, krank, grank, counts, triu)


def _run_group(blkmap, xs, w1, b1, w2, b2):
    grid_spec = pltpu.PrefetchScalarGridSpec(
        num_scalar_prefetch=1,
        grid=(_NBP,),
        in_specs=[
            pl.BlockSpec((_BTG, _D2), lambda b, m: (b, 0)),
            pl.BlockSpec((1, _H, _D), lambda b, m: (m[b], 0, 0)),
            pl.BlockSpec((1, 1, _H), lambda b, m: (m[b], 0, 0)),
            pl.BlockSpec((1, _D, _H), lambda b, m: (m[b], 0, 0)),
            pl.BlockSpec((1, 1, _D), lambda b, m: (m[b], 0, 0)),
        ],
        out_specs=pl.BlockSpec((_BTG, _D2), lambda b, m: (b, 0)),
    )
    return pl.pallas_call(
        _group_body,
        grid_spec=grid_spec,
        out_shape=jax.ShapeDtypeStruct((_NROWS, _D2), jnp.int32),
    )(blkmap, xs, w1, b1.reshape(_E, 1, _H), w2, b2.reshape(_E, 1, _D))


def kernel(x, expert_sel, w1, b1, w2, b2):
    B, S, D = x.shape
    xt = x.reshape(B * S, D)
    gate, krank, grank, counts, xbf = _run_router(xt, expert_sel)
    dsta, dstb, ga, gb, blkmap = _run_prep(gate, krank, grank, counts)
    xs = _sc_dispatch(xbf, dsta.reshape(_T), dstb.reshape(_T))
    ys = _run_group(blkmap.reshape(_NBP), xs, w1, b1, w2, b2)
    ya, yb = _sc_combine(ys, dsta.reshape(_T), dstb.reshape(_T))
    out = _run_add(ya, yb, ga, gb)
    return out.reshape(B, S, D), jnp.array(0.0, dtype=x.dtype)


# speedup vs baseline: 1.2317x; 1.0197x over previous
import functools

import jax
import jax.numpy as jnp
from jax import lax
from jax.experimental import pallas as pl
from jax.experimental.pallas import tpu as pltpu
from jax.experimental.pallas import tpu_sc as plsc

_E = 8
_T = 4096
_D = 1024
_H = 512
_BT = 512
_BTG = 256
_NBP = 40
_NROWS = _NBP * _BTG
_NW = 32
_TPW = _T // _NW
_CD = 64
_CC = 32
_D2 = _D // 2


def _router_body(x_ref, sw_ref, tril_ref, triu_ref, xbf_ref, dsta_ref,
                 dstb_ref, ga_ref, gb_ref, blk_ref, carry, gate_s, krank_s,
                 grank_s):
    i = pl.program_id(0)
    nb = _T // _BT

    @pl.when(i == 0)
    def _init():
        carry[...] = jnp.zeros_like(carry)

    @pl.when(i < nb)
    def _route():
        xl = x_ref[:, :_D2].astype(jnp.bfloat16).astype(jnp.float32)
        xh = x_ref[:, _D2:].astype(jnp.bfloat16).astype(jnp.float32)
        ul = jax.lax.bitcast_convert_type(xl, jnp.uint32) >> 16
        uh = jax.lax.bitcast_convert_type(
            xh, jnp.uint32) & jnp.uint32(0xFFFF0000)
        xbf_ref[...] = jax.lax.bitcast_convert_type(ul | uh, jnp.int32)
        logits = jax.lax.dot_general(
            x_ref[...], sw_ref[...], (((1,), (1,)), ((), ())),
            preferred_element_type=jnp.float32)
        eidx = jax.lax.broadcasted_iota(jnp.int32, logits.shape, 1)
        krank = jnp.zeros_like(logits)
        for j in range(_E):
            vj = logits[:, j:j + 1]
            beats = (vj > logits) | ((vj == logits) & (j < eidx))
            krank += beats.astype(jnp.float32)
        sl = pl.ds(i * _BT, _BT)
        gate_s[sl, :] = jnp.where(krank < 2.0, jax.nn.sigmoid(logits), 0.0)
        krank_s[sl, :] = krank

        ind = (krank < 2.0).astype(jnp.float32)
        local = jax.lax.dot_general(
            tril_ref[...], ind, (((1,), (0,)), ((), ())),
            preferred_element_type=jnp.float32)
        grank_s[sl, :] = local + carry[...]
        carry[...] += jnp.sum(ind, axis=0, keepdims=True)

    @pl.when(i == nb)
    def _prep():
        counts = carry[...]
        pc = jnp.floor((counts + (_BTG - 1.0)) / _BTG)
        offb = jax.lax.dot_general(
            pc, triu_ref[...], (((1,), (0,)), ((), ())),
            preferred_element_type=jnp.float32)

        gate = gate_s[...]
        krank = krank_s[...]
        grank = grank_s[...]
        sel = krank < 2.0
        eidx = jax.lax.broadcasted_iota(jnp.int32, gate.shape, 1).astype(
            jnp.float32)
        e_a = jnp.min(jnp.where(sel, eidx, 99.0), axis=1, keepdims=True)
        e_b = jnp.max(jnp.where(sel, eidx, -1.0), axis=1, keepdims=True)
        m_a = (eidx == e_a).astype(jnp.float32)
        m_b = (eidx == e_b).astype(jnp.float32)
        ga_ref[...] = jnp.sum(gate * m_a, axis=1, keepdims=True)
        gb_ref[...] = jnp.sum(gate * m_b, axis=1, keepdims=True)
        off_a = jnp.sum(offb * m_a, axis=1, keepdims=True) * _BTG
        off_b = jnp.sum(offb * m_b, axis=1, keepdims=True) * _BTG
        r_a = jnp.sum(grank * m_a, axis=1, keepdims=True)
        r_b = jnp.sum(grank * m_b, axis=1, keepdims=True)
        dsta_ref[...] = (off_a + r_a).astype(jnp.int32)
        dstb_ref[...] = (off_b + r_b).astype(jnp.int32)

        b_io = jax.lax.broadcasted_iota(jnp.int32, (_NBP, 1), 0).astype(
            jnp.float32)
        cnt = jnp.sum((b_io >= offb).astype(jnp.float32), axis=1,
                      keepdims=True)
        blk_ref[...] = (cnt - 1.0).astype(jnp.int32)


@functools.lru_cache(maxsize=1)
def _make_sc_dispatch():
    @functools.partial(
        pl.kernel,
        mesh=plsc.VectorSubcoreMesh(core_axis_name="c", subcore_axis_name="s"),
        out_type=jax.ShapeDtypeStruct((_NROWS, _D2), jnp.int32),
        scratch_types=[
            pltpu.VMEM((_CD, _D2), jnp.int32),
            pltpu.VMEM((_CD, _D2), jnp.int32),
            pltpu.VMEM((_CD,), jnp.int32),
            pltpu.VMEM((_CD,), jnp.int32),
            pltpu.VMEM((_CD,), jnp.int32),
            pltpu.VMEM((_CD,), jnp.int32),
            pltpu.SemaphoreType.DMA,
            pltpu.SemaphoreType.DMA,
        ],
    )
    def dispatch(x_hbm, dsta_hbm, dstb_hbm, xs_hbm, rows0, rows1, ia0, ib0,
                 ia1, ib1, sem_l, sem_s):
        rows = (rows0, rows1)
        ia = (ia0, ia1)
        ib = (ib0, ib1)
        wid = lax.axis_index("s") * 2 + lax.axis_index("c")
        base = wid * _TPW
        nch = _TPW // _CD
        pltpu.sync_copy(dsta_hbm.at[pl.ds(base, _CD)], ia0)
        pltpu.sync_copy(dstb_hbm.at[pl.ds(base, _CD)], ib0)
        ld = pltpu.async_copy(x_hbm.at[pl.ds(base, _CD)], rows0, sem_l)
        pend = [None, None]
        for c in range(nch):
            p = c % 2
            q = 1 - p
            ld.wait()
            sa = pltpu.async_copy(rows[p], xs_hbm.at[ia[p]], sem_s)
            sb = pltpu.async_copy(rows[p], xs_hbm.at[ib[p]], sem_s)
            pend[p] = (sa, sb)
            if c + 1 < nch:
                if pend[q] is not None:
                    pend[q][0].wait()
                    pend[q][1].wait()
                    pend[q] = None
                off1 = base + (c + 1) * _CD
                pltpu.sync_copy(dsta_hbm.at[pl.ds(off1, _CD)], ia[q])
                pltpu.sync_copy(dstb_hbm.at[pl.ds(off1, _CD)], ib[q])
                ld = pltpu.async_copy(x_hbm.at[pl.ds(off1, _CD)], rows[q],
                                      sem_l)
        for pp in pend:
            if pp is not None:
                pp[0].wait()
                pp[1].wait()

    return dispatch


def _sc_dispatch(xpk, dsta, dstb):
    return _make_sc_dispatch()(xpk, dsta, dstb)


def _unpack_halves(pk):
    u = jax.lax.bitcast_convert_type(pk, jnp.uint32)
    lo = jax.lax.bitcast_convert_type(u << 16, jnp.float32)
    hi = jax.lax.bitcast_convert_type(u & jnp.uint32(0xFFFF0000), jnp.float32)
    return lo, hi


def _pack_halves(lo, hi):
    ul = jax.lax.bitcast_convert_type(
        lo.astype(jnp.bfloat16).astype(jnp.float32), jnp.uint32) >> 16
    uh = jax.lax.bitcast_convert_type(
        hi.astype(jnp.bfloat16).astype(jnp.float32),
        jnp.uint32) & jnp.uint32(0xFFFF0000)
    return jax.lax.bitcast_convert_type(ul | uh, jnp.int32)


def _group_body(blk_ref, xs_ref, w1_ref, b1_ref, w2_ref, b2_ref, ys_ref):
    xlo, xhi = _unpack_halves(xs_ref[...])
    xb = jnp.concatenate([xlo, xhi], axis=1).astype(jnp.bfloat16)
    h = jax.lax.dot_general(
        xb, w1_ref[0].astype(jnp.bfloat16), (((1,), (1,)), ((), ())),
        preferred_element_type=jnp.float32)
    h = jnp.maximum(h + b1_ref[0], 0.0).astype(jnp.bfloat16)
    y = jax.lax.dot_general(
        h, w2_ref[0].astype(jnp.bfloat16), (((1,), (1,)), ((), ())),
        preferred_element_type=jnp.float32)
    y = y + b2_ref[0]
    ys_ref[...] = _pack_halves(y[:, :_D2], y[:, _D2:])


@functools.lru_cache(maxsize=1)
def _make_sc_combine():
    @functools.partial(
        pl.kernel,
        mesh=plsc.VectorSubcoreMesh(core_axis_name="c", subcore_axis_name="s"),
        out_type=[
            jax.ShapeDtypeStruct((_T, _D2), jnp.int32),
            jax.ShapeDtypeStruct((_T, _D2), jnp.int32),
        ],
        scratch_types=[
            pltpu.VMEM((_CC, _D2), jnp.int32),
            pltpu.VMEM((_CC, _D2), jnp.int32),
            pltpu.VMEM((_CC, _D2), jnp.int32),
            pltpu.VMEM((_CC, _D2), jnp.int32),
            pltpu.VMEM((_CC,), jnp.int32),
            pltpu.VMEM((_CC,), jnp.int32),
            pltpu.VMEM((_CC,), jnp.int32),
            pltpu.VMEM((_CC,), jnp.int32),
            pltpu.SemaphoreType.DMA,
            pltpu.SemaphoreType.DMA,
        ],
    )
    def combine(ys_hbm, dsta_hbm, dstb_hbm, a_hbm, b_hbm, ra0, rb0, ra1, rb1,
                ia0, ib0, ia1, ib1, sem_g, sem_s):
        ra = (ra0, ra1)
        rb = (rb0, rb1)
        ia = (ia0, ia1)
        ib = (ib0, ib1)
        wid = lax.axis_index("s") * 2 + lax.axis_index("c")
        base = wid * _TPW
        nch = _TPW // _CC
        pltpu.sync_copy(dsta_hbm.at[pl.ds(base, _CC)], ia0)
        pltpu.sync_copy(dstb_hbm.at[pl.ds(base, _CC)], ib0)
        ga = pltpu.async_copy(ys_hbm.at[ia0], ra0, sem_g)
        gb = pltpu.async_copy(ys_hbm.at[ib0], rb0, sem_g)
        pend = [None, None]
        for c in range(nch):
            p = c % 2
            q = 1 - p
            off = base + c * _CC
            ga.wait()
            gb.wait()
            sa = pltpu.async_copy(ra[p], a_hbm.at[pl.ds(off, _CC)], sem_s)
            sb = pltpu.async_copy(rb[p], b_hbm.at[pl.ds(off, _CC)], sem_s)
            pend[p] = (sa, sb)
            if c + 1 < nch:
                if pend[q] is not None:
                    pend[q][0].wait()
                    pend[q][1].wait()
                    pend[q] = None
                off1 = base + (c + 1) * _CC
                pltpu.sync_copy(dsta_hbm.at[pl.ds(off1, _CC)], ia[q])
                pltpu.sync_copy(dstb_hbm.at[pl.ds(off1, _CC)], ib[q])
                ga = pltpu.async_copy(ys_hbm.at[ia[q]], ra[q], sem_g)
                gb = pltpu.async_copy(ys_hbm.at[ib[q]], rb[q], sem_g)
        for pp in pend:
            if pp is not None:
                pp[0].wait()
                pp[1].wait()

    return combine


def _sc_combine(ys, dsta, dstb):
    return _make_sc_combine()(ys, dsta, dstb)


def _add_body(a_ref, b_ref, ga_ref, gb_ref, o_ref):
    alo, ahi = _unpack_halves(a_ref[...])
    blo, bhi = _unpack_halves(b_ref[...])
    ga = ga_ref[...]
    gb = gb_ref[...]
    o_ref[...] = jnp.concatenate(
        [ga * alo + gb * blo, ga * ahi + gb * bhi], axis=1)


def _run_add(a, b, ga, gb):
    return pl.pallas_call(
        _add_body,
        grid=(_T // _BT,),
        in_specs=[
            pl.BlockSpec((_BT, _D2), lambda i: (i, 0)),
            pl.BlockSpec((_BT, _D2), lambda i: (i, 0)),
            pl.BlockSpec((_BT, 1), lambda i: (i, 0)),
            pl.BlockSpec((_BT, 1), lambda i: (i, 0)),
        ],
        out_specs=pl.BlockSpec((_BT, _D), lambda i: (i, 0)),
        out_shape=jax.ShapeDtypeStruct((_T, _D), jnp.float32),
    )(a, b, ga, gb)


def _run_router(xt, expert_sel):
    r = jax.lax.broadcasted_iota(jnp.int32, (_BT, _BT), 0)
    c = jax.lax.broadcasted_iota(jnp.int32, (_BT, _BT), 1)
    tril = (c < r).astype(jnp.float32)
    r8 = jax.lax.broadcasted_iota(jnp.int32, (_E, _E), 0)
    c8 = jax.lax.broadcasted_iota(jnp.int32, (_E, _E), 1)
    triu = (r8 < c8).astype(jnp.float32)
    nb = _T // _BT
    last = nb - 1
    return pl.pallas_call(
        _router_body,
        grid=(nb + 1,),
        in_specs=[
            pl.BlockSpec((_BT, _D), lambda i: (jnp.minimum(i, last), 0)),
            pl.BlockSpec((_E, _D), lambda i: (0, 0)),
            pl.BlockSpec((_BT, _BT), lambda i: (0, 0)),
            pl.BlockSpec((_E, _E), lambda i: (0, 0)),
        ],
        out_specs=[
            pl.BlockSpec((_BT, _D2), lambda i: (jnp.minimum(i, last), 0)),
            pl.BlockSpec((_T, 1), lambda i: (0, 0)),
            pl.BlockSpec((_T, 1), lambda i: (0, 0)),
            pl.BlockSpec((_T, 1), lambda i: (0, 0)),
            pl.BlockSpec((_T, 1), lambda i: (0, 0)),
            pl.BlockSpec((_NBP, 1), lambda i: (0, 0)),
        ],
        out_shape=[
            jax.ShapeDtypeStruct((_T, _D2), jnp.int32),
            jax.ShapeDtypeStruct((_T, 1), jnp.int32),
            jax.ShapeDtypeStruct((_T, 1), jnp.int32),
            jax.ShapeDtypeStruct((_T, 1), jnp.float32),
            jax.ShapeDtypeStruct((_T, 1), jnp.float32),
            jax.ShapeDtypeStruct((_NBP, 1), jnp.int32),
        ],
        scratch_shapes=[
            pltpu.VMEM((1, _E), jnp.float32),
            pltpu.VMEM((_T, _E), jnp.float32),
            pltpu.VMEM((_T, _E), jnp.float32),
            pltpu.VMEM((_T, _E), jnp.float32),
        ],
    )(xt, expert_sel, tril, triu)


def _run_group(blkmap, xs, w1, b1, w2, b2):
    grid_spec = pltpu.PrefetchScalarGridSpec(
        num_scalar_prefetch=1,
        grid=(_NBP,),
        in_specs=[
            pl.BlockSpec((_BTG, _D2), lambda b, m: (b, 0)),
            pl.BlockSpec((1, _H, _D), lambda b, m: (m[b], 0, 0)),
            pl.BlockSpec((1, 1, _H), lambda b, m: (m[b], 0, 0)),
            pl.BlockSpec((1, _D, _H), lambda b, m: (m[b], 0, 0)),
            pl.BlockSpec((1, 1, _D), lambda b, m: (m[b], 0, 0)),
        ],
        out_specs=pl.BlockSpec((_BTG, _D2), lambda b, m: (b, 0)),
    )
    return pl.pallas_call(
        _group_body,
        grid_spec=grid_spec,
        out_shape=jax.ShapeDtypeStruct((_NROWS, _D2), jnp.int32),
    )(blkmap, xs, w1, b1.reshape(_E, 1, _H), w2, b2.reshape(_E, 1, _D))


def kernel(x, expert_sel, w1, b1, w2, b2):
    B, S, D = x.shape
    xt = x.reshape(B * S, D)
    xbf, dsta, dstb, ga, gb, blkmap = _run_router(xt, expert_sel)
    xs = _sc_dispatch(xbf, dsta.reshape(_T), dstb.reshape(_T))
    ys = _run_group(blkmap.reshape(_NBP), xs, w1, b1, w2, b2)
    ya, yb = _sc_combine(ys, dsta.reshape(_T), dstb.reshape(_T))
    out = _run_add(ya, yb, ga, gb)
    return out.reshape(B, S, D), jnp.array(0.0, dtype=x.dtype)
